# Initial kernel scaffold; baseline (speedup 1.0000x reference)
import jax
import jax.numpy as jnp
from jax.experimental import pallas as pl


def kernel(x_BSD, biases_N, Wg_DN):
    raise NotImplementedError("write your pallas kernel here")


# speedup vs baseline: 7.8090x; 7.8090x over previous
import functools

import jax
import jax.numpy as jnp
from jax.experimental import pallas as pl
from jax.experimental.pallas import tpu as pltpu

_TM = 1024


def _router_body(x_ref, w_ref, b_ref, mask_ref, s_ref, *, n_exp, k_sel):
    ng = n_exp - 1
    xb = x_ref[...].astype(jnp.bfloat16)
    s_blk = jax.lax.dot_general(
        xb, w_ref[...], (((1,), (0,)), ((), ())),
        preferred_element_type=jnp.float32)
    sT = s_blk.T

    tm = sT.shape[1]
    row = jax.lax.broadcasted_iota(jnp.int32, (n_exp, tm), 0)
    routed = row < ng
    neg = jnp.float32(float("-inf"))

    m = jnp.max(jnp.where(routed, sT, neg), axis=0, keepdims=True)
    e = jnp.where(routed, jnp.exp(sT - m), 0.0)
    ssum = jnp.sum(e, axis=0, keepdims=True)

    bias = b_ref[...]
    sel = jnp.where(routed, e + bias * ssum, neg)

    picked = jnp.zeros((n_exp, tm), dtype=jnp.float32)
    for _ in range(k_sel):
        cur = jnp.max(sel, axis=0, keepdims=True)
        cand = jnp.where(sel == cur, row, n_exp)
        first = jnp.min(cand, axis=0, keepdims=True)
        hit = row == first
        picked += hit.astype(jnp.float32)
        sel = jnp.where(hit, neg, sel)

    ew = e * picked
    ngsum = jnp.sum(ew, axis=0, keepdims=True)
    s_out = jnp.where(routed, ew / ngsum, 1.0)
    mask_ref[...] = jnp.where(routed, picked, 1.0).astype(jnp.int32)
    s_ref[...] = s_out


def kernel(x_BSD, biases_N, Wg_DN):
    b, s, d = x_BSD.shape
    m = b * s
    n = Wg_DN.shape[1]
    x_MD = x_BSD.reshape(m, d)
    bias_N1 = biases_N.reshape(n, 1)

    grid = m // _TM
    body = functools.partial(_router_body, n_exp=n, k_sel=7)
    mask_NM, s_NM = pl.pallas_call(
        body,
        grid=(grid,),
        in_specs=[
            pl.BlockSpec((_TM, d), lambda i: (i, 0)),
            pl.BlockSpec((d, n), lambda i: (0, 0)),
            pl.BlockSpec((n, 1), lambda i: (0, 0)),
        ],
        out_specs=[
            pl.BlockSpec((n, _TM), lambda i: (0, i)),
            pl.BlockSpec((n, _TM), lambda i: (0, i)),
        ],
        out_shape=[
            jax.ShapeDtypeStruct((n, m), jnp.int32),
            jax.ShapeDtypeStruct((n, m), jnp.float32),
        ],
        compiler_params=pltpu.CompilerParams(
            dimension_semantics=("arbitrary",),
        ),
    )(x_MD, Wg_DN, bias_N1)
    return (x_BSD, mask_NM, s_NM)


# speedup vs baseline: 8.2204x; 1.0527x over previous
import functools

import jax
import jax.numpy as jnp
from jax.experimental import pallas as pl
from jax.experimental.pallas import tpu as pltpu

_TM = 2048


def _router_body(x_ref, w_ref, b_ref, mask_ref, s_ref, *, n_exp, k_sel):
    ng = n_exp - 1
    xb = x_ref[...].astype(jnp.bfloat16)
    s_blk = jax.lax.dot_general(
        xb, w_ref[...], (((1,), (0,)), ((), ())),
        preferred_element_type=jnp.float32)
    sT = s_blk.T

    tm = sT.shape[1]
    row = jax.lax.broadcasted_iota(jnp.int32, (n_exp, tm), 0)
    routed = row < ng
    neg = jnp.float32(float("-inf"))

    m = jnp.max(jnp.where(routed, sT, neg), axis=0, keepdims=True)
    e = jnp.where(routed, jnp.exp(sT - m), 0.0)
    ssum = jnp.sum(e, axis=0, keepdims=True)

    bias = b_ref[...]
    sel = jnp.where(routed, e + bias * ssum, neg)

    picked = jnp.zeros((n_exp, tm), dtype=jnp.float32)
    for _ in range(k_sel):
        cur = jnp.max(sel, axis=0, keepdims=True)
        cand = jnp.where(sel == cur, row, n_exp)
        first = jnp.min(cand, axis=0, keepdims=True)
        hit = row == first
        picked += hit.astype(jnp.float32)
        sel = jnp.where(hit, neg, sel)

    ew = e * picked
    ngsum = jnp.sum(ew, axis=0, keepdims=True)
    s_out = jnp.where(routed, ew / ngsum, 1.0)
    mask_ref[...] = jnp.where(routed, picked, 1.0).astype(jnp.int32)
    s_ref[...] = s_out


def kernel(x_BSD, biases_N, Wg_DN):
    b, s, d = x_BSD.shape
    m = b * s
    n = Wg_DN.shape[1]
    x_MD = x_BSD.reshape(m, d)
    bias_N1 = biases_N.reshape(n, 1)

    grid = m // _TM
    body = functools.partial(_router_body, n_exp=n, k_sel=7)
    mask_NM, s_NM = pl.pallas_call(
        body,
        grid=(grid,),
        in_specs=[
            pl.BlockSpec((_TM, d), lambda i: (i, 0)),
            pl.BlockSpec((d, n), lambda i: (0, 0)),
            pl.BlockSpec((n, 1), lambda i: (0, 0)),
        ],
        out_specs=[
            pl.BlockSpec((n, _TM), lambda i: (0, i)),
            pl.BlockSpec((n, _TM), lambda i: (0, i)),
        ],
        out_shape=[
            jax.ShapeDtypeStruct((n, m), jnp.int32),
            jax.ShapeDtypeStruct((n, m), jnp.float32),
        ],
        compiler_params=pltpu.CompilerParams(
            dimension_semantics=("arbitrary",),
        ),
    )(x_MD, Wg_DN, bias_N1)
    return (x_BSD, mask_NM, s_NM)
